# Initial kernel scaffold; baseline (speedup 1.0000x reference)
import jax
import jax.numpy as jnp
from jax.experimental import pallas as pl


def kernel(node_attr, edge_attr, edge_sh, W1, b1, W2, b2, gamma_s, gamma_v, beta_s, edge_index):
    raise NotImplementedError("write your pallas kernel here")


# speedup vs baseline: 3.0756x; 3.0756x over previous
import functools

import jax
import jax.numpy as jnp
import numpy as np
from jax import lax
from jax.experimental import pallas as pl
from jax.experimental.pallas import tpu as pltpu
from jax.experimental.pallas import tpu_sc as plsc

N = 10000
E = 160000
IN = 16
H = 16
EPS = 1e-5
NORM = 1.0 / 4.0

NW = 32
CH = 128
NCH = 40
EPAD = NW * CH * NCH
EPW = CH * NCH
NP = 10112
ROWS_PER_TILE = NP // 16
DUMMY = N

_mesh = plsc.VectorSubcoreMesh(core_axis_name="c", subcore_axis_name="s")
_sc_params = pltpu.CompilerParams(use_tc_tiling_on_sc=False)


@functools.partial(
    pl.kernel,
    out_type=(
        jax.ShapeDtypeStruct((EPAD, IN), jnp.float32),
        jax.ShapeDtypeStruct((2, NP, 16), jnp.float32),
    ),
    mesh=_mesh,
    compiler_params=_sc_params,
    scratch_types=[
        pltpu.VMEM((NCH, CH), jnp.int32),
        pltpu.VMEM((NCH, CH), jnp.int32),
        pltpu.VMEM((CH, IN), jnp.float32),
        pltpu.VMEM((CH, 16), jnp.float32),
        pltpu.VMEM_SHARED((NP, 16), jnp.float32),
        pltpu.SemaphoreType.DMA,
    ],
)
def _gather_count(na_hbm, dst_hbm, src_hbm, zeros16_hbm, ones_hbm,
                  xg_hbm, cnt_hbm, idxd_v, idxs_v, gbuf, onesbuf, cnt_sh, sem):
    c = lax.axis_index("c")
    s = lax.axis_index("s")
    wid = c * 16 + s
    pltpu.sync_copy(dst_hbm.at[wid], idxd_v)
    pltpu.sync_copy(src_hbm.at[wid], idxs_v)
    pltpu.sync_copy(ones_hbm, onesbuf)
    pltpu.sync_copy(zeros16_hbm.at[pl.ds(s * ROWS_PER_TILE, ROWS_PER_TILE)],
                    cnt_sh.at[pl.ds(s * ROWS_PER_TILE, ROWS_PER_TILE)])
    plsc.subcore_barrier()

    def chunk(j, carry):
        pltpu.async_copy(na_hbm.at[idxd_v.at[j]], gbuf, sem).wait()
        pltpu.sync_copy(gbuf, xg_hbm.at[pl.ds(wid * EPW + j * CH, CH)])
        pltpu.sync_copy(onesbuf, cnt_sh.at[idxs_v.at[j]], add=True)
        return carry

    lax.fori_loop(0, NCH, chunk, 0)
    plsc.subcore_barrier()
    pltpu.sync_copy(cnt_sh.at[pl.ds(s * ROWS_PER_TILE, ROWS_PER_TILE)],
                    cnt_hbm.at[c, pl.ds(s * ROWS_PER_TILE, ROWS_PER_TILE)])


@functools.partial(
    pl.kernel,
    out_type=jax.ShapeDtypeStruct((2, NP, 64), jnp.float32),
    mesh=_mesh,
    compiler_params=_sc_params,
    scratch_types=[
        pltpu.VMEM((NCH, CH), jnp.int32),
        pltpu.VMEM((CH, 64), jnp.float32),
        pltpu.VMEM_SHARED((NP, 64), jnp.float32),
        pltpu.SemaphoreType.DMA,
    ],
)
def _scatter_tp(tp_hbm, src_hbm, zeros64_hbm, acc_hbm, idx_v, tpbuf, acc_sh, sem):
    c = lax.axis_index("c")
    s = lax.axis_index("s")
    wid = c * 16 + s
    pltpu.sync_copy(src_hbm.at[wid], idx_v)
    pltpu.sync_copy(zeros64_hbm.at[pl.ds(s * ROWS_PER_TILE, ROWS_PER_TILE)],
                    acc_sh.at[pl.ds(s * ROWS_PER_TILE, ROWS_PER_TILE)])
    plsc.subcore_barrier()

    def chunk(j, carry):
        pltpu.sync_copy(tp_hbm.at[pl.ds(wid * EPW + j * CH, CH)], tpbuf)
        pltpu.sync_copy(tpbuf, acc_sh.at[idx_v.at[j]], add=True)
        return carry

    lax.fori_loop(0, NCH, chunk, 0)
    plsc.subcore_barrier()
    pltpu.sync_copy(acc_sh.at[pl.ds(s * ROWS_PER_TILE, ROWS_PER_TILE)],
                    acc_hbm.at[c, pl.ds(s * ROWS_PER_TILE, ROWS_PER_TILE)])


BE = 2048


def _tp_body(ea_ref, xg_ref, sh_ref, W1_ref, b1_ref, Rm_ref, Tm_ref,
             CP_ref, B2P_ref, Q_ref, out_ref):
    h = jnp.maximum(ea_ref[...] @ W1_ref[...] + b1_ref[...], 0.0)
    z = (h @ Rm_ref[...]) * (xg_ref[...] @ Tm_ref[...])
    a64 = z @ CP_ref[...] + xg_ref[...] @ B2P_ref[...]
    out_ref[...] = a64 * (sh_ref[...] @ Q_ref[...])


def _fin_body(acc_ref, cnt_ref, na_ref, gs_ref, gv_ref, bs_ref,
              G3_ref, Pv_ref, out_ref):
    summed = acc_ref[0, :N, :] + acc_ref[1, :N, :]
    cnt = cnt_ref[0, :N, :] + cnt_ref[1, :N, :]
    div = jnp.maximum(cnt[:, 0:1], 1.0)
    out = summed / div
    s = out[:, :16] + na_ref[...]
    v = out[:, 16:]
    mu = jnp.mean(s, axis=0, keepdims=True)
    sc = s - mu
    s_norm = jnp.mean(sc * sc, axis=0, keepdims=True)
    s_out = sc * (gs_ref[...] * lax.rsqrt(s_norm + EPS)) + bs_ref[...]
    m48 = jnp.mean(v * v, axis=0, keepdims=True)
    vn16 = m48 @ G3_ref[...]
    scale48 = (gv_ref[...] * lax.rsqrt(vn16 + EPS)) @ Pv_ref[...]
    out_ref[...] = jnp.concatenate([s_out, v * scale48], axis=1)


def _const_spec(shape):
    nd = len(shape)
    return pl.BlockSpec(shape, lambda i, _nd=nd: (0,) * _nd)


@jax.jit
def kernel(node_attr, edge_attr, edge_sh, W1, b1, W2, b2,
           gamma_s, gamma_v, beta_s, edge_index):
    f32 = jnp.float32
    pad = EPAD - E
    ea = jnp.pad(edge_attr, ((0, pad), (0, 0)))
    sh = jnp.pad(edge_sh, ((0, pad), (0, 0)))
    dst3 = jnp.pad(edge_index[1], (0, pad)).reshape(NW, NCH, CH)
    src3 = jnp.pad(edge_index[0], (0, pad),
                   constant_values=DUMMY).reshape(NW, NCH, CH)
    zeros16 = jnp.zeros((NP, 16), f32)
    zeros64 = jnp.zeros((NP, 64), f32)
    ones128 = jnp.ones((CH, 16), f32)

    eye16 = jnp.eye(16, dtype=f32)
    Rm = jnp.repeat(eye16, 16, axis=1)
    Tm = jnp.tile(eye16, (1, 16))
    Cs = W2[:, :256].reshape(256, 16)
    Cv = W2[:, 256:].reshape(256, 16)
    Cm = jnp.concatenate([Cs, Cv], axis=1)
    B2s = b2[:256].reshape(16, 16)
    B2v = b2[256:].reshape(16, 16)
    B2m = jnp.concatenate([B2s, B2v], axis=1)
    Pv = jnp.repeat(eye16, 3, axis=1)
    Pm = jnp.zeros((32, 64), f32).at[:16, :16].set(eye16).at[16:, 16:].set(Pv)
    Qm = jnp.zeros((4, 64), f32).at[0, :16].set(1.0)
    Qm = Qm.at[1:, 16:].set(jnp.tile(jnp.eye(3, dtype=f32), (1, 16)))
    CP = (Cm @ Pm) * NORM
    B2P = (B2m @ Pm) * NORM
    G3 = Pv.T / 3.0

    xg, cnt = _gather_count(node_attr, dst3, src3, zeros16, ones128)

    tp = pl.pallas_call(
        _tp_body,
        grid=(EPAD // BE,),
        in_specs=[
            pl.BlockSpec((BE, IN), lambda i: (i, 0)),
            pl.BlockSpec((BE, IN), lambda i: (i, 0)),
            pl.BlockSpec((BE, 4), lambda i: (i, 0)),
            _const_spec((H, H)),
            _const_spec((1, H)),
            _const_spec((H, 256)),
            _const_spec((IN, 256)),
            _const_spec((256, 64)),
            _const_spec((IN, 64)),
            _const_spec((4, 64)),
        ],
        out_specs=pl.BlockSpec((BE, 64), lambda i: (i, 0)),
        out_shape=jax.ShapeDtypeStruct((EPAD, 64), f32),
    )(ea, xg, sh, W1, b1[None, :], Rm, Tm, CP, B2P, Qm)

    acc = _scatter_tp(tp, src3, zeros64)

    out = pl.pallas_call(
        _fin_body,
        out_shape=jax.ShapeDtypeStruct((N, 64), f32),
    )(acc, cnt, node_attr, gamma_s[None, :], gamma_v[None, :],
      beta_s[None, :], G3, Pv)
    return out
